# Initial kernel scaffold; baseline (speedup 1.0000x reference)
import functools
import jax
import jax.numpy as jnp
from jax import lax
from jax.experimental import pallas as pl
from jax.experimental.pallas import tpu as pltpu
from jax.experimental.pallas import tpu_sc as plsc

CH = 128
DEG_W = 16


def _sc_mesh():
    info = plsc.get_sparse_core_info()
    return (plsc.VectorSubcoreMesh(core_axis_name="c", subcore_axis_name="s"),
            info.num_cores, info.num_subcores)


def _make_deg_kernel(EP, NP, NC, NS, mesh):
    EW = EP // (NC * NS)
    n_chunks = EW // CH
    RP = NP // NS

    @functools.partial(
        pl.kernel,
        out_type=jax.ShapeDtypeStruct((NC, NP, DEG_W), jnp.float32),
        mesh=mesh,
        scratch_types=[
            pltpu.VMEM((CH,), jnp.int32),
            pltpu.VMEM((CH, DEG_W), jnp.float32),
            pltpu.VMEM_SHARED((NP, DEG_W), jnp.float32),
        ],
    )
    def deg_kernel(col_hbm, out_hbm, idxc, buf, acc):
        cid = lax.axis_index("c")
        sid = lax.axis_index("s")
        wid = sid * NC + cid

        def zero_row(i, _):
            buf[i, :] = jnp.zeros((DEG_W,), jnp.float32)
            return 0
        lax.fori_loop(0, CH, zero_row, 0)
        base = sid * RP
        off = 0
        while off < RP:
            m = min(CH, RP - off)
            pltpu.sync_copy(buf.at[pl.ds(0, m)], acc.at[pl.ds(base + off, m)])
            off += m
        def one_row(i, _):
            buf[i, :] = jnp.where(lax.iota(jnp.int32, 16) == 0, 1.0, 0.0)
            return 0
        lax.fori_loop(0, CH, one_row, 0)
        plsc.subcore_barrier()

        def edge_chunk(c, _):
            e0 = wid * EW + c * CH
            pltpu.sync_copy(col_hbm.at[pl.ds(e0, CH)], idxc)
            pltpu.sync_copy(buf, acc.at[idxc], add=True)
            return 0
        lax.fori_loop(0, n_chunks, edge_chunk, 0)
        plsc.subcore_barrier()

        off = 0
        while off < RP:
            m = min(CH, RP - off)
            pltpu.sync_copy(acc.at[pl.ds(base + off, m)], buf.at[pl.ds(0, m)])
            pltpu.sync_copy(buf.at[pl.ds(0, m)],
                            out_hbm.at[cid, pl.ds(base + off, m)])
            off += m

    return deg_kernel


def _make_spmm_kernel(EP, NP, D, NC, NS, mesh):
    EW = EP // (NC * NS)
    n_chunks = EW // CH
    RP = NP // NS

    @functools.partial(
        pl.kernel,
        out_type=jax.ShapeDtypeStruct((NC, NP, D), jnp.float32),
        mesh=mesh,
        scratch_types=[
            pltpu.VMEM((CH,), jnp.int32),
            pltpu.VMEM((CH,), jnp.int32),
            pltpu.VMEM((CH, D), jnp.float32),
            pltpu.VMEM_SHARED((NP, D), jnp.float32),
            pltpu.SemaphoreType.DMA,
        ],
    )
    def spmm_kernel(g_hbm, row_hbm, col_hbm, out_hbm, idxr, idxc, buf, acc, sem):
        cid = lax.axis_index("c")
        sid = lax.axis_index("s")
        wid = sid * NC + cid

        def zero_row(i, _):
            for j in range(D // 16):
                buf[i, pl.ds(j * 16, 16)] = jnp.zeros((16,), jnp.float32)
            return 0
        lax.fori_loop(0, CH, zero_row, 0)
        base = sid * RP
        off = 0
        while off < RP:
            m = min(CH, RP - off)
            pltpu.sync_copy(buf.at[pl.ds(0, m)], acc.at[pl.ds(base + off, m)])
            off += m
        plsc.subcore_barrier()

        def edge_chunk(c, _):
            e0 = wid * EW + c * CH
            pltpu.sync_copy(row_hbm.at[pl.ds(e0, CH)], idxr)
            pltpu.sync_copy(col_hbm.at[pl.ds(e0, CH)], idxc)
            pltpu.async_copy(g_hbm.at[idxr], buf, sem).wait()
            pltpu.sync_copy(buf, acc.at[idxc], add=True)
            return 0
        lax.fori_loop(0, n_chunks, edge_chunk, 0)
        plsc.subcore_barrier()

        off = 0
        while off < RP:
            m = min(CH, RP - off)
            pltpu.sync_copy(acc.at[pl.ds(base + off, m)], buf.at[pl.ds(0, m)])
            pltpu.sync_copy(buf.at[pl.ds(0, m)],
                            out_hbm.at[cid, pl.ds(base + off, m)])
            off += m

    return spmm_kernel


def _tc_prep(degp_ref, x_ref, dinv_ref, g_ref):
    N = x_ref.shape[0]
    deg = degp_ref[0] + degp_ref[1]
    deg = deg[:, 0:1] + 1.0
    dinv = lax.rsqrt(deg)
    dinv_ref[...] = dinv
    g_ref[...] = x_ref[...] * dinv[:N]


def _tc_layer(sp_ref, g_ref, dinv_ref, w_ref, b_ref, gam_ref, bet_ref,
              out_ref, *, relu):
    N = g_ref.shape[0]
    s = sp_ref[0, :N] + sp_ref[1, :N] + g_ref[...]
    agg = s * dinv_ref[:N]
    z = jnp.dot(agg, w_ref[...], preferred_element_type=jnp.float32) + b_ref[...]
    mu = jnp.mean(z, axis=0, keepdims=True)
    var = jnp.mean((z - mu) * (z - mu), axis=0, keepdims=True)
    h = gam_ref[...] * (z - mu) * lax.rsqrt(var + 1e-5) + bet_ref[...]
    if relu:
        h = jnp.maximum(h, 0.0)
    out_ref[...] = h * dinv_ref[:N]


def _tc_final(sp_ref, g_ref, dinv_ref, w_ref, b_ref, gam_ref, bet_ref,
              wc_ref, bc_ref, out_ref):
    N = g_ref.shape[0]
    s = sp_ref[0, :N] + sp_ref[1, :N] + g_ref[...]
    agg = s * dinv_ref[:N]
    z = jnp.dot(agg, w_ref[...], preferred_element_type=jnp.float32) + b_ref[...]
    mu = jnp.mean(z, axis=0, keepdims=True)
    var = jnp.mean((z - mu) * (z - mu), axis=0, keepdims=True)
    h = gam_ref[...] * (z - mu) * lax.rsqrt(var + 1e-5) + bet_ref[...]
    out_ref[...] = (jnp.dot(h, wc_ref[...], preferred_element_type=jnp.float32)
                    + bc_ref[...])


@jax.jit
def kernel(x, edge_index, W1, b1, g1, be1, W2, b2, g2, be2, Wc, bc):
    N, D = x.shape
    O = Wc.shape[1]
    E = edge_index.shape[1]

    mesh, NC, NS = _sc_mesh()
    NW = NC * NS
    NP = ((N + 1 + NS - 1) // NS) * NS
    EW = ((E + NW * CH - 1) // (NW * CH)) * CH
    EP = EW * NW

    pad = EP - E
    row = jnp.concatenate([edge_index[0], jnp.zeros((pad,), jnp.int32)])
    col = jnp.concatenate([edge_index[1], jnp.full((pad,), N, jnp.int32)])

    deg_k = _make_deg_kernel(EP, NP, NC, NS, mesh)
    spmm_k = _make_spmm_kernel(EP, NP, D, NC, NS, mesh)

    degp = deg_k(col)

    dinv, gx = pl.pallas_call(
        _tc_prep,
        out_shape=(jax.ShapeDtypeStruct((NP, 1), jnp.float32),
                   jax.ShapeDtypeStruct((N, D), jnp.float32)),
    )(degp, x)

    s1 = spmm_k(gx, row, col)
    g2x = pl.pallas_call(
        functools.partial(_tc_layer, relu=True),
        out_shape=jax.ShapeDtypeStruct((N, D), jnp.float32),
    )(s1, gx, dinv, W1, b1, g1, be1)

    s2 = spmm_k(g2x, row, col)
    out = pl.pallas_call(
        _tc_final,
        out_shape=jax.ShapeDtypeStruct((N, O), jnp.float32),
    )(s2, g2x, dinv, W2, b2, g2, be2, Wc, bc)
    return out


# speedup vs baseline: 11.2348x; 11.2348x over previous
import functools
import jax
import jax.numpy as jnp
from jax import lax
from jax.experimental import pallas as pl
from jax.experimental.pallas import tpu as pltpu
from jax.experimental.pallas import tpu_sc as plsc

CH = 128
DEG_W = 16


def _sc_mesh():
    info = plsc.get_sparse_core_info()
    return (plsc.VectorSubcoreMesh(core_axis_name="c", subcore_axis_name="s"),
            info.num_cores, info.num_subcores)


def _make_deg_kernel(EP, NP, NC, NS, mesh):
    EW = EP // (NC * NS)
    n_chunks = EW // CH
    RP = NP // NS

    @functools.partial(
        pl.kernel,
        out_type=jax.ShapeDtypeStruct((NC, NP, DEG_W), jnp.float32),
        mesh=mesh,
        scratch_types=[
            pltpu.VMEM((CH,), jnp.int32),
            pltpu.VMEM((CH, DEG_W), jnp.float32),
            pltpu.VMEM_SHARED((NP, DEG_W), jnp.float32),
        ],
    )
    def deg_kernel(col_hbm, out_hbm, idxc, buf, acc):
        cid = lax.axis_index("c")
        sid = lax.axis_index("s")
        wid = sid * NC + cid

        def zero_row(i, _):
            buf[i, :] = jnp.zeros((DEG_W,), jnp.float32)
            return 0
        lax.fori_loop(0, CH, zero_row, 0)
        base = sid * RP
        off = 0
        while off < RP:
            m = min(CH, RP - off)
            pltpu.sync_copy(buf.at[pl.ds(0, m)], acc.at[pl.ds(base + off, m)])
            off += m
        def one_row(i, _):
            buf[i, :] = jnp.where(lax.iota(jnp.int32, 16) == 0, 1.0, 0.0)
            return 0
        lax.fori_loop(0, CH, one_row, 0)
        plsc.subcore_barrier()

        def edge_chunk(c, _):
            e0 = wid * EW + c * CH
            pltpu.sync_copy(col_hbm.at[pl.ds(e0, CH)], idxc)
            pltpu.sync_copy(buf, acc.at[idxc], add=True)
            return 0
        lax.fori_loop(0, n_chunks, edge_chunk, 0)
        plsc.subcore_barrier()

        off = 0
        while off < RP:
            m = min(CH, RP - off)
            pltpu.sync_copy(acc.at[pl.ds(base + off, m)], buf.at[pl.ds(0, m)])
            pltpu.sync_copy(buf.at[pl.ds(0, m)],
                            out_hbm.at[cid, pl.ds(base + off, m)])
            off += m

    return deg_kernel


def _make_spmm_kernel(EP, NP, D, NC, NS, mesh):
    EW = EP // (NC * NS)
    n_chunks = EW // CH
    RP = NP // NS

    @functools.partial(
        pl.kernel,
        out_type=jax.ShapeDtypeStruct((NC, NP, D), jnp.float32),
        mesh=mesh,
        scratch_types=[
            pltpu.VMEM((CH,), jnp.int32),
            pltpu.VMEM((CH,), jnp.int32),
            pltpu.VMEM((CH, D), jnp.float32),
            pltpu.VMEM_SHARED((NP, D), jnp.float32),
            pltpu.SemaphoreType.DMA,
        ],
    )
    def spmm_kernel(g_hbm, row_hbm, col_hbm, out_hbm, idxr, idxc, buf, acc, sem):
        cid = lax.axis_index("c")
        sid = lax.axis_index("s")
        wid = sid * NC + cid

        def zero_row(i, _):
            for j in range(D // 16):
                buf[i, pl.ds(j * 16, 16)] = jnp.zeros((16,), jnp.float32)
            return 0
        lax.fori_loop(0, CH, zero_row, 0)
        base = sid * RP
        off = 0
        while off < RP:
            m = min(CH, RP - off)
            pltpu.sync_copy(buf.at[pl.ds(0, m)], acc.at[pl.ds(base + off, m)])
            off += m
        plsc.subcore_barrier()

        def edge_chunk(c, _):
            e0 = wid * EW + c * CH
            pltpu.sync_copy(row_hbm.at[pl.ds(e0, CH)], idxr)
            pltpu.sync_copy(col_hbm.at[pl.ds(e0, CH)], idxc)
            pltpu.async_copy(g_hbm.at[idxr], buf, sem).wait()
            pltpu.sync_copy(buf, acc.at[idxc], add=True)
            return 0
        lax.fori_loop(0, n_chunks, edge_chunk, 0)
        plsc.subcore_barrier()

        off = 0
        while off < RP:
            m = min(CH, RP - off)
            pltpu.sync_copy(acc.at[pl.ds(base + off, m)], buf.at[pl.ds(0, m)])
            pltpu.sync_copy(buf.at[pl.ds(0, m)],
                            out_hbm.at[cid, pl.ds(base + off, m)])
            off += m

    return spmm_kernel


def _tc_prep(degp_ref, x_ref, dinv_ref, g_ref):
    N = x_ref.shape[0]
    deg = degp_ref[0] + degp_ref[1]
    deg = deg[:, 0:1] + 1.0
    dinv = lax.rsqrt(deg)
    dinv_ref[...] = dinv
    g_ref[...] = x_ref[...] * dinv[:N]


def _tc_layer(sp_ref, g_ref, dinv_ref, w_ref, b_ref, gam_ref, bet_ref,
              out_ref, *, relu):
    N = g_ref.shape[0]
    s = sp_ref[0, :N] + sp_ref[1, :N] + g_ref[...]
    agg = s * dinv_ref[:N]
    z = jnp.dot(agg, w_ref[...], preferred_element_type=jnp.float32) + b_ref[...]
    mu = jnp.mean(z, axis=0, keepdims=True)
    var = jnp.mean((z - mu) * (z - mu), axis=0, keepdims=True)
    h = gam_ref[...] * (z - mu) * lax.rsqrt(var + 1e-5) + bet_ref[...]
    if relu:
        h = jnp.maximum(h, 0.0)
    out_ref[...] = h * dinv_ref[:N]


def _tc_final(sp_ref, g_ref, dinv_ref, w_ref, b_ref, gam_ref, bet_ref,
              wc_ref, bc_ref, out_ref):
    N = g_ref.shape[0]
    s = sp_ref[0, :N] + sp_ref[1, :N] + g_ref[...]
    agg = s * dinv_ref[:N]
    z = jnp.dot(agg, w_ref[...], preferred_element_type=jnp.float32) + b_ref[...]
    mu = jnp.mean(z, axis=0, keepdims=True)
    var = jnp.mean((z - mu) * (z - mu), axis=0, keepdims=True)
    h = gam_ref[...] * (z - mu) * lax.rsqrt(var + 1e-5) + bet_ref[...]
    out_ref[...] = (jnp.dot(h, wc_ref[...], preferred_element_type=jnp.float32)
                    + bc_ref[...])


@jax.jit
def kernel(x, edge_index, W1, b1, g1, be1, W2, b2, g2, be2, Wc, bc):
    N, D = x.shape
    O = Wc.shape[1]
    E = edge_index.shape[1]

    mesh, NC, NS = _sc_mesh()
    NW = NC * NS
    NP = ((N + 1 + 127) // 128) * 128
    EW = ((E + NW * CH - 1) // (NW * CH)) * CH
    EP = EW * NW

    pad = EP - E
    row = jnp.concatenate([edge_index[0], jnp.zeros((pad,), jnp.int32)])
    col = jnp.concatenate([edge_index[1], jnp.full((pad,), N, jnp.int32)])

    deg_k = _make_deg_kernel(EP, NP, NC, NS, mesh)
    spmm_k = _make_spmm_kernel(EP, NP, D, NC, NS, mesh)

    degp = deg_k(col)

    dinv, gx = pl.pallas_call(
        _tc_prep,
        out_shape=(jax.ShapeDtypeStruct((NP, 1), jnp.float32),
                   jax.ShapeDtypeStruct((N, D), jnp.float32)),
    )(degp, x)

    s1 = spmm_k(gx, row, col)
    g2x = pl.pallas_call(
        functools.partial(_tc_layer, relu=True),
        out_shape=jax.ShapeDtypeStruct((N, D), jnp.float32),
    )(s1, gx, dinv, W1, b1, g1, be1)

    s2 = spmm_k(g2x, row, col)
    out = pl.pallas_call(
        _tc_final,
        out_shape=jax.ShapeDtypeStruct((N, O), jnp.float32),
    )(s2, g2x, dinv, W2, b2, g2, be2, Wc, bc)
    return out
